# baseline (device time: 34243 ns/iter reference)
import jax
import jax.numpy as jnp
from jax import lax
from jax.experimental import pallas as pl
from jax.experimental.pallas import tpu as pltpu


def kernel(Q, K, V):
    b, sq, h, d = Q.shape
    _, skv, _, _ = K.shape
    scale = d ** -0.5
    cw = 128

    def body(q_ref, k_ref, v_ref, o_ref, comm_ref, send_sems, recv_sems):
        i = pl.program_id(0)
        my_x = lax.axis_index("x")
        my_y = lax.axis_index("y")
        my_z = lax.axis_index("z")
        peer = (1 - my_x, my_y, my_z)

        @pl.when(i == 0)
        def _():
            barrier_sem = pltpu.get_barrier_semaphore()
            pl.semaphore_signal(
                barrier_sem, inc=1, device_id=peer,
                device_id_type=pl.DeviceIdType.MESH,
            )
            pl.semaphore_wait(barrier_sem, 1)

        qb = q_ref[0, 0]
        k2 = k_ref[0].reshape(skv * h, d)
        v2 = v_ref[0].reshape(skv * h, d)

        c = lax.dot_general(qb, k2, (((1,), (1,)), ((), ())))
        jh = lax.broadcasted_iota(jnp.int32, (h, skv * h), 1) % h
        hh = lax.broadcasted_iota(jnp.int32, (h, skv * h), 0)
        cm = jnp.where(jh == hh, c * scale, -1e30)
        mb = jnp.max(cm, axis=-1, keepdims=True)
        p = jnp.exp(cm - mb)
        lb = jnp.sum(p, axis=-1, keepdims=True)
        ob = lax.dot_general(p, v2, (((1,), (0,)), ((), ())))

        comm_ref[0, i, :, 0:d] = ob
        comm_ref[0, i, :, d : d + 1] = mb
        comm_ref[0, i, :, d + 1 : d + 2] = lb

        rdma = pltpu.make_async_remote_copy(
            src_ref=comm_ref.at[0, i],
            dst_ref=comm_ref.at[1, i],
            send_sem=send_sems.at[i],
            recv_sem=recv_sems.at[i],
            device_id=peer,
            device_id_type=pl.DeviceIdType.MESH,
        )
        rdma.start()

        @pl.when(i == b - 1)
        def _():
            for j in range(b):
                w = pltpu.make_async_remote_copy(
                    src_ref=comm_ref.at[0, j],
                    dst_ref=comm_ref.at[1, j],
                    send_sem=send_sems.at[j],
                    recv_sem=recv_sems.at[j],
                    device_id=peer,
                    device_id_type=pl.DeviceIdType.MESH,
                )
                w.wait()

            o1 = comm_ref[0, :, :, 0:d]
            m1 = comm_ref[0, :, :, d : d + 1][:, :, 0]
            l1 = comm_ref[0, :, :, d + 1 : d + 2][:, :, 0]
            o2 = comm_ref[1, :, :, 0:d]
            m2 = comm_ref[1, :, :, d : d + 1][:, :, 0]
            l2 = comm_ref[1, :, :, d + 1 : d + 2][:, :, 0]

            mg = jnp.maximum(m1, m2)
            ca = jnp.exp(m1 - mg)
            cb = jnp.exp(m2 - mg)
            lg = l1 * ca + l2 * cb
            og = (o1 * ca[:, :, None] + o2 * cb[:, :, None]) / lg[:, :, None]
            o_ref[:, 0, :, :] = og

    return pl.pallas_call(
        body,
        grid=(b,),
        out_shape=jax.ShapeDtypeStruct((b, sq, h, d), jnp.float32),
        in_specs=[
            pl.BlockSpec((1, sq, h, d), lambda i: (i, 0, 0, 0)),
            pl.BlockSpec((1, skv, h, d), lambda i: (i, 0, 0, 0)),
            pl.BlockSpec((1, skv, h, d), lambda i: (i, 0, 0, 0)),
        ],
        out_specs=pl.BlockSpec((b, sq, h, d), lambda i: (0, 0, 0, 0)),
        scratch_shapes=[
            pltpu.VMEM((2, b, h, cw), jnp.float32),
            pltpu.SemaphoreType.DMA((b,)),
            pltpu.SemaphoreType.DMA((b,)),
        ],
        compiler_params=pltpu.CompilerParams(
            collective_id=0,
            dimension_semantics=("arbitrary",),
        ),
    )(Q, K, V)


# device time: 17891 ns/iter; 1.9140x vs baseline; 1.9140x over previous
import jax
import jax.numpy as jnp
from jax import lax
from jax.experimental import pallas as pl
from jax.experimental.pallas import tpu as pltpu


def kernel(Q, K, V):
    b, sq, h, d = Q.shape
    _, skv, _, _ = K.shape
    scale = d ** -0.5
    cw = 128

    Kt = jnp.transpose(K, (0, 2, 3, 1))
    Vt = jnp.transpose(V, (0, 2, 3, 1))

    def body(q_ref, k_ref, v_ref, o_ref, comm_ref, send_sems, recv_sems):
        i = pl.program_id(0)
        my_x = lax.axis_index("x")
        my_y = lax.axis_index("y")
        my_z = lax.axis_index("z")
        peer = (1 - my_x, my_y, my_z)

        @pl.when(i == 0)
        def _():
            barrier_sem = pltpu.get_barrier_semaphore()
            pl.semaphore_signal(
                barrier_sem, inc=1, device_id=peer,
                device_id_type=pl.DeviceIdType.MESH,
            )
            pl.semaphore_wait(barrier_sem, 1)

        qb = q_ref[0, 0]
        k2 = k_ref[0].reshape(h * d, skv)
        v2 = v_ref[0].reshape(h * d, skv)

        hh = lax.broadcasted_iota(jnp.int32, (h, h), 0)
        hh2 = lax.broadcasted_iota(jnp.int32, (h, h), 1)
        eye_h = (hh == hh2).astype(jnp.float32)
        qbd = (qb[:, None, :] * eye_h[:, :, None]).reshape(h, h * d)

        s = lax.dot_general(qbd, k2, (((1,), (0,)), ((), ()))) * scale
        mb = jnp.max(s, axis=-1, keepdims=True)
        p = jnp.exp(s - mb)
        lb = jnp.sum(p, axis=-1, keepdims=True)
        g = lax.dot_general(p, v2, (((1,), (1,)), ((), ())))
        ob = jnp.sum(g.reshape(h, h, d) * eye_h[:, :, None], axis=1)

        comm_ref[0, i, :, 0:d] = ob
        comm_ref[0, i, :, d : d + 1] = mb
        comm_ref[0, i, :, d + 1 : d + 2] = lb

        rdma = pltpu.make_async_remote_copy(
            src_ref=comm_ref.at[0, i],
            dst_ref=comm_ref.at[1, i],
            send_sem=send_sems.at[i],
            recv_sem=recv_sems.at[i],
            device_id=peer,
            device_id_type=pl.DeviceIdType.MESH,
        )
        rdma.start()

        @pl.when(i == b - 1)
        def _():
            for j in range(b):
                w = pltpu.make_async_remote_copy(
                    src_ref=comm_ref.at[0, j],
                    dst_ref=comm_ref.at[1, j],
                    send_sem=send_sems.at[j],
                    recv_sem=recv_sems.at[j],
                    device_id=peer,
                    device_id_type=pl.DeviceIdType.MESH,
                )
                w.wait()

            o1 = comm_ref[0, :, :, 0:d]
            m1 = comm_ref[0, :, :, d : d + 1][:, :, 0]
            l1 = comm_ref[0, :, :, d + 1 : d + 2][:, :, 0]
            o2 = comm_ref[1, :, :, 0:d]
            m2 = comm_ref[1, :, :, d : d + 1][:, :, 0]
            l2 = comm_ref[1, :, :, d + 1 : d + 2][:, :, 0]

            mg = jnp.maximum(m1, m2)
            ca = jnp.exp(m1 - mg)
            cb = jnp.exp(m2 - mg)
            lg = l1 * ca + l2 * cb
            og = (o1 * ca[:, :, None] + o2 * cb[:, :, None]) / lg[:, :, None]
            o_ref[:, 0, :, :] = og

    return pl.pallas_call(
        body,
        grid=(b,),
        out_shape=jax.ShapeDtypeStruct((b, sq, h, d), jnp.float32),
        in_specs=[
            pl.BlockSpec((1, sq, h, d), lambda i: (i, 0, 0, 0)),
            pl.BlockSpec((1, h, d, skv), lambda i: (i, 0, 0, 0)),
            pl.BlockSpec((1, h, d, skv), lambda i: (i, 0, 0, 0)),
        ],
        out_specs=pl.BlockSpec((b, sq, h, d), lambda i: (0, 0, 0, 0)),
        scratch_shapes=[
            pltpu.VMEM((2, b, h, cw), jnp.float32),
            pltpu.SemaphoreType.DMA((b,)),
            pltpu.SemaphoreType.DMA((b,)),
        ],
        compiler_params=pltpu.CompilerParams(
            collective_id=0,
            dimension_semantics=("arbitrary",),
        ),
    )(Q, Kt, Vt)


# device time: 15580 ns/iter; 2.1979x vs baseline; 1.1483x over previous
import jax
import jax.numpy as jnp
from jax import lax
from jax.experimental import pallas as pl
from jax.experimental.pallas import tpu as pltpu


def kernel(Q, K, V):
    b, sq, h, d = Q.shape
    _, skv, _, _ = K.shape
    scale = d ** -0.5
    cw = 128

    Kt = jnp.transpose(K, (0, 2, 3, 1))
    Vt = jnp.transpose(V, (0, 2, 3, 1))
    Kt = pltpu.with_memory_space_constraint(Kt, pltpu.MemorySpace.HBM)
    Vt = pltpu.with_memory_space_constraint(Vt, pltpu.MemorySpace.HBM)

    def body(q_ref, k_ref, v_ref, o_ref, comm_ref, send_sems, recv_sems):
        i = pl.program_id(0)
        my_x = lax.axis_index("x")
        my_y = lax.axis_index("y")
        my_z = lax.axis_index("z")
        peer = (1 - my_x, my_y, my_z)

        @pl.when(i == 0)
        def _():
            barrier_sem = pltpu.get_barrier_semaphore()
            pl.semaphore_signal(
                barrier_sem, inc=1, device_id=peer,
                device_id_type=pl.DeviceIdType.MESH,
            )
            pl.semaphore_wait(barrier_sem, 1)

        qb = q_ref[0, 0]
        k2 = k_ref[0].reshape(h * d, skv)
        v2 = v_ref[0].reshape(h * d, skv)

        hh = lax.broadcasted_iota(jnp.int32, (h, h), 0)
        hh2 = lax.broadcasted_iota(jnp.int32, (h, h), 1)
        eye_h = (hh == hh2).astype(jnp.float32)
        qbd = (qb[:, None, :] * eye_h[:, :, None]).reshape(h, h * d)

        s = lax.dot_general(qbd, k2, (((1,), (0,)), ((), ()))) * scale
        mb = jnp.max(s, axis=-1, keepdims=True)
        p = jnp.exp(s - mb)
        lb = jnp.sum(p, axis=-1, keepdims=True)
        g = lax.dot_general(p, v2, (((1,), (1,)), ((), ())))
        ob = jnp.sum(g.reshape(h, h, d) * eye_h[:, :, None], axis=1)

        comm_ref[0, i, :, 0:d] = ob
        comm_ref[0, i, :, d : d + 1] = mb
        comm_ref[0, i, :, d + 1 : d + 2] = lb

        rdma = pltpu.make_async_remote_copy(
            src_ref=comm_ref.at[0, i],
            dst_ref=comm_ref.at[1, i],
            send_sem=send_sems.at[i],
            recv_sem=recv_sems.at[i],
            device_id=peer,
            device_id_type=pl.DeviceIdType.MESH,
        )
        rdma.start()

        @pl.when(i == b - 1)
        def _():
            for j in range(b):
                w = pltpu.make_async_remote_copy(
                    src_ref=comm_ref.at[0, j],
                    dst_ref=comm_ref.at[1, j],
                    send_sem=send_sems.at[j],
                    recv_sem=recv_sems.at[j],
                    device_id=peer,
                    device_id_type=pl.DeviceIdType.MESH,
                )
                w.wait()

            o1 = comm_ref[0, :, :, 0:d]
            m1 = comm_ref[0, :, :, d : d + 1][:, :, 0]
            l1 = comm_ref[0, :, :, d + 1 : d + 2][:, :, 0]
            o2 = comm_ref[1, :, :, 0:d]
            m2 = comm_ref[1, :, :, d : d + 1][:, :, 0]
            l2 = comm_ref[1, :, :, d + 1 : d + 2][:, :, 0]

            mg = jnp.maximum(m1, m2)
            ca = jnp.exp(m1 - mg)
            cb = jnp.exp(m2 - mg)
            lg = l1 * ca + l2 * cb
            og = (o1 * ca[:, :, None] + o2 * cb[:, :, None]) / lg[:, :, None]
            o_ref[:, 0, :, :] = og

    return pl.pallas_call(
        body,
        grid=(b,),
        out_shape=jax.ShapeDtypeStruct((b, sq, h, d), jnp.float32),
        in_specs=[
            pl.BlockSpec((1, sq, h, d), lambda i: (i, 0, 0, 0)),
            pl.BlockSpec((1, h, d, skv), lambda i: (i, 0, 0, 0)),
            pl.BlockSpec((1, h, d, skv), lambda i: (i, 0, 0, 0)),
        ],
        out_specs=pl.BlockSpec((b, sq, h, d), lambda i: (0, 0, 0, 0)),
        scratch_shapes=[
            pltpu.VMEM((2, b, h, cw), jnp.float32),
            pltpu.SemaphoreType.DMA((b,)),
            pltpu.SemaphoreType.DMA((b,)),
        ],
        compiler_params=pltpu.CompilerParams(
            collective_id=0,
            dimension_semantics=("arbitrary",),
        ),
    )(Q, Kt, Vt)


# device time: 14845 ns/iter; 2.3067x vs baseline; 1.0495x over previous
import jax
import jax.numpy as jnp
from jax import lax
from jax.experimental import pallas as pl
from jax.experimental.pallas import tpu as pltpu


def kernel(Q, K, V):
    b, sq, h, d = Q.shape
    _, skv, _, _ = K.shape
    scale = d ** -0.5
    cw = 128

    Kt = jnp.transpose(K, (0, 2, 3, 1))
    Vt = jnp.transpose(V, (0, 2, 3, 1))
    Kt = pltpu.with_memory_space_constraint(Kt, pltpu.MemorySpace.HBM)
    Vt = pltpu.with_memory_space_constraint(Vt, pltpu.MemorySpace.HBM)
    Qh = pltpu.with_memory_space_constraint(Q, pltpu.MemorySpace.HBM)

    def body(q_ref, k_ref, v_ref, o_ref, comm_ref, send_sems, recv_sems):
        i = pl.program_id(0)
        my_x = lax.axis_index("x")
        my_y = lax.axis_index("y")
        my_z = lax.axis_index("z")
        peer = (1 - my_x, my_y, my_z)

        @pl.when(i == 0)
        def _():
            barrier_sem = pltpu.get_barrier_semaphore()
            pl.semaphore_signal(
                barrier_sem, inc=1, device_id=peer,
                device_id_type=pl.DeviceIdType.MESH,
            )
            pl.semaphore_wait(barrier_sem, 1)

        qb = q_ref[0, 0]
        k2 = k_ref[0].reshape(h * d, skv)
        v2 = v_ref[0].reshape(h * d, skv)

        hh = lax.broadcasted_iota(jnp.int32, (h, h), 0)
        hh2 = lax.broadcasted_iota(jnp.int32, (h, h), 1)
        eye_h = (hh == hh2).astype(jnp.float32)
        qbd = (qb[:, None, :] * eye_h[:, :, None]).reshape(h, h * d)

        s = lax.dot_general(qbd, k2, (((1,), (0,)), ((), ()))) * scale
        mb = jnp.max(s, axis=-1, keepdims=True)
        p = jnp.exp(s - mb)
        lb = jnp.sum(p, axis=-1, keepdims=True)
        g = lax.dot_general(p, v2, (((1,), (1,)), ((), ())))
        ob = jnp.sum(g.reshape(h, h, d) * eye_h[:, :, None], axis=1)

        comm_ref[0, i, :, 0:d] = ob
        comm_ref[0, i, :, d : d + 1] = mb
        comm_ref[0, i, :, d + 1 : d + 2] = lb

        rdma = pltpu.make_async_remote_copy(
            src_ref=comm_ref.at[0, i],
            dst_ref=comm_ref.at[1, i],
            send_sem=send_sems.at[i],
            recv_sem=recv_sems.at[i],
            device_id=peer,
            device_id_type=pl.DeviceIdType.MESH,
        )
        rdma.start()

        @pl.when(i == b - 1)
        def _():
            for j in range(b):
                w = pltpu.make_async_remote_copy(
                    src_ref=comm_ref.at[0, j],
                    dst_ref=comm_ref.at[1, j],
                    send_sem=send_sems.at[j],
                    recv_sem=recv_sems.at[j],
                    device_id=peer,
                    device_id_type=pl.DeviceIdType.MESH,
                )
                w.wait()

            o1 = comm_ref[0, :, :, 0:d]
            m1 = comm_ref[0, :, :, d : d + 1][:, :, 0]
            l1 = comm_ref[0, :, :, d + 1 : d + 2][:, :, 0]
            o2 = comm_ref[1, :, :, 0:d]
            m2 = comm_ref[1, :, :, d : d + 1][:, :, 0]
            l2 = comm_ref[1, :, :, d + 1 : d + 2][:, :, 0]

            mg = jnp.maximum(m1, m2)
            ca = jnp.exp(m1 - mg)
            cb = jnp.exp(m2 - mg)
            lg = l1 * ca + l2 * cb
            og = (o1 * ca[:, :, None] + o2 * cb[:, :, None]) / lg[:, :, None]
            o_ref[:, 0, :, :] = og

    return pl.pallas_call(
        body,
        grid=(b,),
        out_shape=jax.ShapeDtypeStruct((b, sq, h, d), jnp.float32),
        in_specs=[
            pl.BlockSpec((1, sq, h, d), lambda i: (i, 0, 0, 0)),
            pl.BlockSpec((1, h, d, skv), lambda i: (i, 0, 0, 0)),
            pl.BlockSpec((1, h, d, skv), lambda i: (i, 0, 0, 0)),
        ],
        out_specs=pl.BlockSpec((b, sq, h, d), lambda i: (0, 0, 0, 0)),
        scratch_shapes=[
            pltpu.VMEM((2, b, h, cw), jnp.float32),
            pltpu.SemaphoreType.DMA((b,)),
            pltpu.SemaphoreType.DMA((b,)),
        ],
        compiler_params=pltpu.CompilerParams(
            collective_id=0,
            dimension_semantics=("arbitrary",),
        ),
    )(Qh, Kt, Vt)


# device time: 12323 ns/iter; 2.7788x vs baseline; 1.2047x over previous
import jax
import jax.numpy as jnp
from jax import lax
from jax.experimental import pallas as pl
from jax.experimental.pallas import tpu as pltpu


def kernel(Q, K, V):
    b, sq, h, d = Q.shape
    _, skv, _, _ = K.shape
    scale = d ** -0.5
    cw = 128
    BB = 2
    nsteps = b // BB

    Kt = jnp.transpose(K, (0, 2, 3, 1))
    Vt = jnp.transpose(V, (0, 2, 3, 1))
    Kt = pltpu.with_memory_space_constraint(Kt, pltpu.MemorySpace.HBM)
    Vt = pltpu.with_memory_space_constraint(Vt, pltpu.MemorySpace.HBM)
    Qh = pltpu.with_memory_space_constraint(Q, pltpu.MemorySpace.HBM)

    def body(q_ref, k_ref, v_ref, o_ref, comm_ref, send_sems, recv_sems):
        i = pl.program_id(0)
        my_x = lax.axis_index("x")
        my_y = lax.axis_index("y")
        my_z = lax.axis_index("z")
        peer = (1 - my_x, my_y, my_z)

        @pl.when(i == 0)
        def _():
            barrier_sem = pltpu.get_barrier_semaphore()
            pl.semaphore_signal(
                barrier_sem, inc=1, device_id=peer,
                device_id_type=pl.DeviceIdType.MESH,
            )
            pl.semaphore_wait(barrier_sem, 1)

        hh = lax.broadcasted_iota(jnp.int32, (h, h), 0)
        hh2 = lax.broadcasted_iota(jnp.int32, (h, h), 1)
        eye_h = (hh == hh2).astype(jnp.float32)

        for j in range(BB):
            qb = q_ref[j, 0]
            k2 = k_ref[j].reshape(h * d, skv)
            v2 = v_ref[j].reshape(h * d, skv)

            qbd = (qb[:, None, :] * eye_h[:, :, None]).reshape(h, h * d)

            s = lax.dot_general(qbd, k2, (((1,), (0,)), ((), ()))) * scale
            mb = jnp.max(s, axis=-1, keepdims=True)
            p = jnp.exp(s - mb)
            lb = jnp.sum(p, axis=-1, keepdims=True)
            g = lax.dot_general(p, v2, (((1,), (1,)), ((), ())))
            ob = jnp.sum(g.reshape(h, h, d) * eye_h[:, :, None], axis=1)

            bi = i * BB + j
            comm_ref[0, bi, :, 0:d] = ob
            comm_ref[0, bi, :, d : d + 1] = mb
            comm_ref[0, bi, :, d + 1 : d + 2] = lb

        rdma = pltpu.make_async_remote_copy(
            src_ref=comm_ref.at[0, pl.ds(i * BB, BB)],
            dst_ref=comm_ref.at[1, pl.ds(i * BB, BB)],
            send_sem=send_sems.at[i],
            recv_sem=recv_sems.at[i],
            device_id=peer,
            device_id_type=pl.DeviceIdType.MESH,
        )
        rdma.start()

        @pl.when(i == nsteps - 1)
        def _():
            for j in range(nsteps):
                w = pltpu.make_async_remote_copy(
                    src_ref=comm_ref.at[0, pl.ds(j * BB, BB)],
                    dst_ref=comm_ref.at[1, pl.ds(j * BB, BB)],
                    send_sem=send_sems.at[j],
                    recv_sem=recv_sems.at[j],
                    device_id=peer,
                    device_id_type=pl.DeviceIdType.MESH,
                )
                w.wait()

            o1 = comm_ref[0, :, :, 0:d]
            m1 = comm_ref[0, :, :, d : d + 1][:, :, 0]
            l1 = comm_ref[0, :, :, d + 1 : d + 2][:, :, 0]
            o2 = comm_ref[1, :, :, 0:d]
            m2 = comm_ref[1, :, :, d : d + 1][:, :, 0]
            l2 = comm_ref[1, :, :, d + 1 : d + 2][:, :, 0]

            mg = jnp.maximum(m1, m2)
            ca = jnp.exp(m1 - mg)
            cb = jnp.exp(m2 - mg)
            lg = l1 * ca + l2 * cb
            og = (o1 * ca[:, :, None] + o2 * cb[:, :, None]) / lg[:, :, None]
            o_ref[:, 0, :, :] = og

    return pl.pallas_call(
        body,
        grid=(nsteps,),
        out_shape=jax.ShapeDtypeStruct((b, sq, h, d), jnp.float32),
        in_specs=[
            pl.BlockSpec((BB, sq, h, d), lambda i: (i, 0, 0, 0)),
            pl.BlockSpec((BB, h, d, skv), lambda i: (i, 0, 0, 0)),
            pl.BlockSpec((BB, h, d, skv), lambda i: (i, 0, 0, 0)),
        ],
        out_specs=pl.BlockSpec((b, sq, h, d), lambda i: (0, 0, 0, 0)),
        scratch_shapes=[
            pltpu.VMEM((2, b, h, cw), jnp.float32),
            pltpu.SemaphoreType.DMA((nsteps,)),
            pltpu.SemaphoreType.DMA((nsteps,)),
        ],
        compiler_params=pltpu.CompilerParams(
            collective_id=0,
            dimension_semantics=("arbitrary",),
        ),
    )(Qh, Kt, Vt)


# device time: 12119 ns/iter; 2.8256x vs baseline; 1.0168x over previous
import jax
import jax.numpy as jnp
from jax import lax
from jax.experimental import pallas as pl
from jax.experimental.pallas import tpu as pltpu


def kernel(Q, K, V):
    b, sq, h, d = Q.shape
    _, skv, _, _ = K.shape
    scale = d ** -0.5
    cw = 128
    BB = 4
    nsteps = b // BB

    Kt = jnp.transpose(K, (0, 2, 3, 1))
    Vt = jnp.transpose(V, (0, 2, 3, 1))
    Kt = pltpu.with_memory_space_constraint(Kt, pltpu.MemorySpace.HBM)
    Vt = pltpu.with_memory_space_constraint(Vt, pltpu.MemorySpace.HBM)
    Qh = pltpu.with_memory_space_constraint(Q, pltpu.MemorySpace.HBM)

    def body(q_ref, k_ref, v_ref, o_ref, comm_ref, send_sems, recv_sems):
        i = pl.program_id(0)
        my_x = lax.axis_index("x")
        my_y = lax.axis_index("y")
        my_z = lax.axis_index("z")
        peer = (1 - my_x, my_y, my_z)

        @pl.when(i == 0)
        def _():
            barrier_sem = pltpu.get_barrier_semaphore()
            pl.semaphore_signal(
                barrier_sem, inc=1, device_id=peer,
                device_id_type=pl.DeviceIdType.MESH,
            )
            pl.semaphore_wait(barrier_sem, 1)

        hh = lax.broadcasted_iota(jnp.int32, (h, h), 0)
        hh2 = lax.broadcasted_iota(jnp.int32, (h, h), 1)
        eye_h = (hh == hh2).astype(jnp.float32)

        for j in range(BB):
            qb = q_ref[j, 0]
            k2 = k_ref[j].reshape(h * d, skv)
            v2 = v_ref[j].reshape(h * d, skv)

            qbd = (qb[:, None, :] * eye_h[:, :, None]).reshape(h, h * d)

            s = lax.dot_general(qbd, k2, (((1,), (0,)), ((), ()))) * scale
            mb = jnp.max(s, axis=-1, keepdims=True)
            p = jnp.exp(s - mb)
            lb = jnp.sum(p, axis=-1, keepdims=True)
            g = lax.dot_general(p, v2, (((1,), (1,)), ((), ())))
            ob = jnp.sum(g.reshape(h, h, d) * eye_h[:, :, None], axis=1)

            bi = i * BB + j
            comm_ref[0, bi, :, 0:d] = ob
            comm_ref[0, bi, :, d : d + 1] = mb
            comm_ref[0, bi, :, d + 1 : d + 2] = lb

        rdma = pltpu.make_async_remote_copy(
            src_ref=comm_ref.at[0, pl.ds(i * BB, BB)],
            dst_ref=comm_ref.at[1, pl.ds(i * BB, BB)],
            send_sem=send_sems.at[i],
            recv_sem=recv_sems.at[i],
            device_id=peer,
            device_id_type=pl.DeviceIdType.MESH,
        )
        rdma.start()

        @pl.when(i == nsteps - 1)
        def _():
            for j in range(nsteps):
                w = pltpu.make_async_remote_copy(
                    src_ref=comm_ref.at[0, pl.ds(j * BB, BB)],
                    dst_ref=comm_ref.at[1, pl.ds(j * BB, BB)],
                    send_sem=send_sems.at[j],
                    recv_sem=recv_sems.at[j],
                    device_id=peer,
                    device_id_type=pl.DeviceIdType.MESH,
                )
                w.wait()

            o1 = comm_ref[0, :, :, 0:d]
            m1 = comm_ref[0, :, :, d : d + 1][:, :, 0]
            l1 = comm_ref[0, :, :, d + 1 : d + 2][:, :, 0]
            o2 = comm_ref[1, :, :, 0:d]
            m2 = comm_ref[1, :, :, d : d + 1][:, :, 0]
            l2 = comm_ref[1, :, :, d + 1 : d + 2][:, :, 0]

            mg = jnp.maximum(m1, m2)
            ca = jnp.exp(m1 - mg)
            cb = jnp.exp(m2 - mg)
            lg = l1 * ca + l2 * cb
            og = (o1 * ca[:, :, None] + o2 * cb[:, :, None]) / lg[:, :, None]
            o_ref[:, 0, :, :] = og

    return pl.pallas_call(
        body,
        grid=(nsteps,),
        out_shape=jax.ShapeDtypeStruct((b, sq, h, d), jnp.float32),
        in_specs=[
            pl.BlockSpec((BB, sq, h, d), lambda i: (i, 0, 0, 0)),
            pl.BlockSpec((BB, h, d, skv), lambda i: (i, 0, 0, 0)),
            pl.BlockSpec((BB, h, d, skv), lambda i: (i, 0, 0, 0)),
        ],
        out_specs=pl.BlockSpec((b, sq, h, d), lambda i: (0, 0, 0, 0)),
        scratch_shapes=[
            pltpu.VMEM((2, b, h, cw), jnp.float32),
            pltpu.SemaphoreType.DMA((nsteps,)),
            pltpu.SemaphoreType.DMA((nsteps,)),
        ],
        compiler_params=pltpu.CompilerParams(
            collective_id=0,
            dimension_semantics=("arbitrary",),
        ),
    )(Qh, Kt, Vt)
